# Initial kernel scaffold; baseline (speedup 1.0000x reference)
import jax
import jax.numpy as jnp
from jax.experimental import pallas as pl


def kernel(loc_preds, loc_targets, cls_preds, cls_targets):
    raise NotImplementedError("write your pallas kernel here")


# speedup vs baseline: 2.0725x; 2.0725x over previous
import functools

import jax
import jax.numpy as jnp
from jax.experimental import pallas as pl
from jax.experimental.pallas import tpu as pltpu


def _body(lp_ref, lt_ref, ct_ref, sum_ref, cnt_ref):
    c = pl.program_id(0)

    @pl.when(c == 0)
    def _init():
        sum_ref[0, 0] = 0.0
        cnt_ref[0, 0] = 0.0

    d = lp_ref[...] - lt_ref[...]
    ad = jnp.abs(d)
    sl1 = jnp.where(ad < 1.0, 0.5 * d * d, ad - 0.5)
    per_anchor = jnp.sum(sl1, axis=2)
    pos = ct_ref[...] > 0
    masked = jnp.where(pos, per_anchor, 0.0)
    sum_ref[0, 0] += jnp.sum(masked)
    cnt_ref[0, 0] += jnp.sum(pos.astype(jnp.float32))


@functools.partial(jax.jit, static_argnames=("interpret",))
def _ohem(loc_preds, loc_targets, cls_targets, interpret=False):
    B, A, L = loc_preds.shape
    CH = 512
    grid = (A // CH,)
    ct = cls_targets.astype(jnp.int32)
    s, n = pl.pallas_call(
        _body,
        grid=grid,
        in_specs=[
            pl.BlockSpec((B, CH, L), lambda c: (0, c, 0)),
            pl.BlockSpec((B, CH, L), lambda c: (0, c, 0)),
            pl.BlockSpec((B, CH), lambda c: (0, c)),
        ],
        out_specs=[
            pl.BlockSpec(memory_space=pltpu.SMEM),
            pl.BlockSpec(memory_space=pltpu.SMEM),
        ],
        out_shape=[
            jax.ShapeDtypeStruct((1, 1), jnp.float32),
            jax.ShapeDtypeStruct((1, 1), jnp.float32),
        ],
        interpret=interpret,
    )(loc_preds, loc_targets, ct)
    return 0.2 * s[0, 0] / n[0, 0]


def kernel(loc_preds, loc_targets, cls_preds, cls_targets):
    return _ohem(loc_preds, loc_targets, cls_targets)


# speedup vs baseline: 3.3898x; 1.6356x over previous
import functools

import jax
import jax.numpy as jnp
from jax.experimental import pallas as pl
from jax.experimental.pallas import tpu as pltpu


def _body(lp_ref, lt_ref, ct_ref, sum_ref, cnt_ref):
    c = pl.program_id(0)

    @pl.when(c == 0)
    def _init():
        sum_ref[0, 0] = 0.0
        cnt_ref[0, 0] = 0.0

    d = lp_ref[...] - lt_ref[...]
    ad = jnp.abs(d)
    sl1 = jnp.where(ad < 1.0, 0.5 * d * d, ad - 0.5)
    B, W = sl1.shape
    per_anchor = jnp.sum(sl1.reshape(B, W // 8, 8), axis=2)
    pos = ct_ref[...] > 0
    masked = jnp.where(pos, per_anchor, 0.0)
    sum_ref[0, 0] += jnp.sum(masked)
    cnt_ref[0, 0] += jnp.sum(pos.astype(jnp.float32))


@functools.partial(jax.jit, static_argnames=("interpret",))
def _ohem(loc_preds, loc_targets, cls_targets, interpret=False):
    B, A, L = loc_preds.shape
    lp = loc_preds.reshape(B, A * L)
    lt = loc_targets.reshape(B, A * L)
    CH = 2048
    W = CH * L
    grid = (A // CH,)
    ct = cls_targets.astype(jnp.int32)
    s, n = pl.pallas_call(
        _body,
        grid=grid,
        in_specs=[
            pl.BlockSpec((B, W), lambda c: (0, c)),
            pl.BlockSpec((B, W), lambda c: (0, c)),
            pl.BlockSpec((B, CH), lambda c: (0, c)),
        ],
        out_specs=[
            pl.BlockSpec(memory_space=pltpu.SMEM),
            pl.BlockSpec(memory_space=pltpu.SMEM),
        ],
        out_shape=[
            jax.ShapeDtypeStruct((1, 1), jnp.float32),
            jax.ShapeDtypeStruct((1, 1), jnp.float32),
        ],
        interpret=interpret,
    )(lp, lt, ct)
    return 0.2 * s[0, 0] / n[0, 0]


def kernel(loc_preds, loc_targets, cls_preds, cls_targets):
    return _ohem(loc_preds, loc_targets, cls_targets)


# speedup vs baseline: 6.0391x; 1.7816x over previous
import functools

import jax
import jax.numpy as jnp
from jax.experimental import pallas as pl
from jax.experimental.pallas import tpu as pltpu


def _body(lp_ref, lt_ref, m8_ref, ct_ref, sum_ref, cnt_ref, acc_ref, pacc_ref):
    c = pl.program_id(0)
    nsteps = pl.num_programs(0)

    @pl.when(c == 0)
    def _init():
        acc_ref[...] = jnp.zeros_like(acc_ref)
        pacc_ref[...] = jnp.zeros_like(pacc_ref)

    d = lp_ref[...] - lt_ref[...]
    ad = jnp.abs(d)
    sl1 = jnp.where(ad < 1.0, 0.5 * d * d, ad - 0.5)
    masked = sl1 * m8_ref[...].astype(jnp.float32)
    acc_ref[...] += masked
    pacc_ref[...] += jnp.where(ct_ref[...] > 0, 1.0, 0.0)

    @pl.when(c == nsteps - 1)
    def _finish():
        sum_ref[0, 0] = jnp.sum(acc_ref[...])
        cnt_ref[0, 0] = jnp.sum(pacc_ref[...])


@functools.partial(jax.jit, static_argnames=("interpret",))
def _ohem(loc_preds, loc_targets, cls_targets, interpret=False):
    B, A, L = loc_preds.shape
    lp = loc_preds.reshape(B, A * L)
    lt = loc_targets.reshape(B, A * L)
    CH = 2048
    W = CH * L
    grid = (A // CH,)
    ct = cls_targets.astype(jnp.int32)
    m8 = jnp.repeat(ct.astype(jnp.int8), L, axis=1)
    s, n = pl.pallas_call(
        _body,
        grid=grid,
        in_specs=[
            pl.BlockSpec((B, W), lambda c: (0, c)),
            pl.BlockSpec((B, W), lambda c: (0, c)),
            pl.BlockSpec((B, W), lambda c: (0, c)),
            pl.BlockSpec((B, CH), lambda c: (0, c)),
        ],
        out_specs=[
            pl.BlockSpec(memory_space=pltpu.SMEM),
            pl.BlockSpec(memory_space=pltpu.SMEM),
        ],
        out_shape=[
            jax.ShapeDtypeStruct((1, 1), jnp.float32),
            jax.ShapeDtypeStruct((1, 1), jnp.float32),
        ],
        scratch_shapes=[
            pltpu.VMEM((B, W), jnp.float32),
            pltpu.VMEM((B, CH), jnp.float32),
        ],
        interpret=interpret,
    )(lp, lt, m8, ct)
    return 0.2 * s[0, 0] / n[0, 0]


def kernel(loc_preds, loc_targets, cls_preds, cls_targets):
    return _ohem(loc_preds, loc_targets, cls_targets)


# speedup vs baseline: 59.1752x; 9.7986x over previous
import functools

import jax
import jax.numpy as jnp
from jax.experimental import pallas as pl
from jax.experimental.pallas import tpu as pltpu


def _body(lp_ref, lt_ref, ct_ref, sum_ref, cnt_ref, acc_ref, pacc_ref):
    c = pl.program_id(0)
    nsteps = pl.num_programs(0)

    @pl.when(c == 0)
    def _init():
        acc_ref[...] = jnp.zeros_like(acc_ref)
        pacc_ref[...] = jnp.zeros_like(pacc_ref)

    d = lp_ref[...] - lt_ref[...]
    ad = jnp.abs(d)
    sl1 = jnp.where(ad < 1.0, 0.5 * d * d, ad - 0.5)
    pos = (ct_ref[...] > 0).astype(jnp.float32)
    acc_ref[...] += sl1 * pos[:, None, :]
    pacc_ref[...] += pos

    @pl.when(c == nsteps - 1)
    def _finish():
        sum_ref[0, 0] = jnp.sum(acc_ref[...])
        cnt_ref[0, 0] = jnp.sum(pacc_ref[...])


@functools.partial(jax.jit, static_argnames=("interpret",))
def _ohem(loc_preds, loc_targets, cls_targets, interpret=False):
    B, A, L = loc_preds.shape
    lpT = jnp.transpose(loc_preds, (0, 2, 1))
    ltT = jnp.transpose(loc_targets, (0, 2, 1))
    CH = 2048
    grid = (A // CH,)
    ct = cls_targets.astype(jnp.int32)
    s, n = pl.pallas_call(
        _body,
        grid=grid,
        in_specs=[
            pl.BlockSpec((B, L, CH), lambda c: (0, 0, c)),
            pl.BlockSpec((B, L, CH), lambda c: (0, 0, c)),
            pl.BlockSpec((B, CH), lambda c: (0, c)),
        ],
        out_specs=[
            pl.BlockSpec(memory_space=pltpu.SMEM),
            pl.BlockSpec(memory_space=pltpu.SMEM),
        ],
        out_shape=[
            jax.ShapeDtypeStruct((1, 1), jnp.float32),
            jax.ShapeDtypeStruct((1, 1), jnp.float32),
        ],
        scratch_shapes=[
            pltpu.VMEM((B, L, CH), jnp.float32),
            pltpu.VMEM((B, CH), jnp.float32),
        ],
        interpret=interpret,
    )(lpT, ltT, ct)
    return 0.2 * s[0, 0] / n[0, 0]


def kernel(loc_preds, loc_targets, cls_preds, cls_targets):
    return _ohem(loc_preds, loc_targets, cls_targets)


# speedup vs baseline: 66.5011x; 1.1238x over previous
import functools

import jax
import jax.numpy as jnp
from jax.experimental import pallas as pl
from jax.experimental.pallas import tpu as pltpu


def _body(lp_ref, lt_ref, ct_ref, sum_ref, cnt_ref, acc_ref, pacc_ref):
    c = pl.program_id(0)
    nsteps = pl.num_programs(0)

    @pl.when(c == 0)
    def _init():
        acc_ref[...] = jnp.zeros_like(acc_ref)
        pacc_ref[...] = jnp.zeros_like(pacc_ref)

    d = lp_ref[...] - lt_ref[...]
    ad = jnp.abs(d)
    sl1 = jnp.where(ad < 1.0, 0.5 * d * d, ad - 0.5)
    pos = (ct_ref[...] > 0).astype(jnp.float32)
    acc_ref[...] += sl1 * pos[:, None, :]
    pacc_ref[...] += pos

    @pl.when(c == nsteps - 1)
    def _finish():
        sum_ref[0, 0] = jnp.sum(acc_ref[...])
        cnt_ref[0, 0] = jnp.sum(pacc_ref[...])


@functools.partial(jax.jit, static_argnames=("interpret",))
def _ohem(loc_preds, loc_targets, cls_targets, interpret=False):
    B, A, L = loc_preds.shape
    lpT = jnp.transpose(loc_preds, (0, 2, 1))
    ltT = jnp.transpose(loc_targets, (0, 2, 1))
    CH = 4096
    grid = (A // CH,)
    ct = cls_targets.astype(jnp.int32)
    s, n = pl.pallas_call(
        _body,
        grid=grid,
        in_specs=[
            pl.BlockSpec((B, L, CH), lambda c: (0, 0, c)),
            pl.BlockSpec((B, L, CH), lambda c: (0, 0, c)),
            pl.BlockSpec((B, CH), lambda c: (0, c)),
        ],
        out_specs=[
            pl.BlockSpec(memory_space=pltpu.SMEM),
            pl.BlockSpec(memory_space=pltpu.SMEM),
        ],
        out_shape=[
            jax.ShapeDtypeStruct((1, 1), jnp.float32),
            jax.ShapeDtypeStruct((1, 1), jnp.float32),
        ],
        scratch_shapes=[
            pltpu.VMEM((B, L, CH), jnp.float32),
            pltpu.VMEM((B, CH), jnp.float32),
        ],
        interpret=interpret,
    )(lpT, ltT, ct)
    return 0.2 * s[0, 0] / n[0, 0]


def kernel(loc_preds, loc_targets, cls_preds, cls_targets):
    return _ohem(loc_preds, loc_targets, cls_targets)
